# Initial kernel scaffold; baseline (speedup 1.0000x reference)
import jax
import jax.numpy as jnp
from jax.experimental import pallas as pl


def kernel(rpn_probs, rpn_bbox, anchors):
    raise NotImplementedError("write your pallas kernel here")


# speedup vs baseline: 11.3506x; 11.3506x over previous
import functools

import jax
import jax.numpy as jnp
from jax.experimental import pallas as pl
from jax.experimental.pallas import tpu as pltpu

_PROPOSAL_COUNT = 1000
_SCORE_THRES = 0.5
_PRE_NMS = 6000
_NMS_THR = 0.7
_NEG = -1e9


def _nms_kernel(score_ref, d0_ref, d1_ref, d2_ref, d3_ref,
                a0_ref, a1_ref, a2_ref, a3_ref,
                oy1_ref, ox1_ref, oy2_ref, ox2_ref,
                sc_ref, y1s_ref, x1s_ref, y2s_ref, x2s_ref, ar_ref):
    score = score_ref[...]
    B, N = score.shape
    bits = jax.lax.bitcast_convert_type(score, jnp.int32)
    iot = jax.lax.broadcasted_iota(jnp.int32, (B, N), 1)

    lo0 = jnp.zeros((B, 1), jnp.int32)
    hi0 = jnp.full((B, 1), 0x7F800000, jnp.int32)

    def bs_val(_, lh):
        lo, hi = lh
        mid = lo + ((hi - lo) >> 1)
        cnt = jnp.sum((bits >= mid).astype(jnp.int32), axis=1, keepdims=True)
        ge = cnt >= _PRE_NMS
        return jnp.where(ge, mid, lo), jnp.where(ge, hi, mid)

    vstar, _ = jax.lax.fori_loop(0, 31, bs_val, (lo0, hi0))

    gt = bits > vstar
    tie = bits == vstar
    c_gt = jnp.sum(gt.astype(jnp.int32), axis=1, keepdims=True)
    need = _PRE_NMS - c_gt

    lo1 = jnp.full((B, 1), -1, jnp.int32)
    hi1 = jnp.full((B, 1), N - 1, jnp.int32)

    def bs_idx(_, lh):
        lo, hi = lh
        mid = lo + ((hi - lo) >> 1)
        cnt = jnp.sum((tie & (iot <= mid)).astype(jnp.int32), axis=1,
                      keepdims=True)
        ge = cnt >= need
        return jnp.where(ge, lo, mid), jnp.where(ge, mid, hi)

    _, istar = jax.lax.fori_loop(0, 15, bs_idx, (lo1, hi1))

    elig = gt | (tie & (iot <= istar))

    a0 = a0_ref[...]
    a1 = a1_ref[...]
    a2 = a2_ref[...]
    a3 = a3_ref[...]
    dy = d0_ref[...] * 0.1
    dx = d1_ref[...] * 0.1
    dh = d2_ref[...] * 0.2
    dw = d3_ref[...] * 0.2
    h = a2 - a0
    w = a3 - a1
    cy = a0 + 0.5 * h
    cx = a1 + 0.5 * w
    cy = cy + dy * h
    cx = cx + dx * w
    h = h * jnp.exp(dh)
    w = w * jnp.exp(dw)
    y1 = cy - 0.5 * h
    x1 = cx - 0.5 * w
    y2 = y1 + h
    x2 = x1 + w
    y1 = jnp.clip(y1, 0.0, 1.0)
    x1 = jnp.clip(x1, 0.0, 1.0)
    y2 = jnp.clip(y2, 0.0, 1.0)
    x2 = jnp.clip(x2, 0.0, 1.0)

    y1s_ref[...] = y1
    x1s_ref[...] = x1
    y2s_ref[...] = y2
    x2s_ref[...] = x2
    ar_ref[...] = (y2 - y1) * (x2 - x1)
    sc_ref[...] = jnp.where(elig & (score >= _SCORE_THRES), score, _NEG)

    lane = jax.lax.broadcasted_iota(jnp.int32, (B, 128), 1)

    def step(j, accs):
        ay1, ax1, ay2, ax2 = accs
        sc = sc_ref[...]
        y1 = y1s_ref[...]
        x1 = x1s_ref[...]
        y2 = y2s_ref[...]
        x2 = x2s_ref[...]
        areas = ar_ref[...]
        best = jnp.max(sc, axis=1, keepdims=True)
        idx = jnp.min(jnp.where(sc == best, iot, N), axis=1, keepdims=True)
        onehot = iot == idx
        z = jnp.zeros_like(sc)
        by1 = jnp.sum(jnp.where(onehot, y1, z), axis=1, keepdims=True)
        bx1 = jnp.sum(jnp.where(onehot, x1, z), axis=1, keepdims=True)
        by2 = jnp.sum(jnp.where(onehot, y2, z), axis=1, keepdims=True)
        bx2 = jnp.sum(jnp.where(onehot, x2, z), axis=1, keepdims=True)
        barea = (by2 - by1) * (bx2 - bx1)
        yy1 = jnp.maximum(by1, y1)
        xx1 = jnp.maximum(bx1, x1)
        yy2 = jnp.minimum(by2, y2)
        xx2 = jnp.minimum(bx2, x2)
        inter = jnp.maximum(yy2 - yy1, 0.0) * jnp.maximum(xx2 - xx1, 0.0)
        union = barea + areas - inter
        iou = inter / (union + 1e-8)
        nsc = jnp.where(iou > _NMS_THR, _NEG, sc)
        nsc = jnp.where(onehot, _NEG, nsc)
        sc_ref[...] = nsc
        valid = best > (_NEG / 2.0)
        zc = jnp.zeros_like(by1)
        sel = lane == j
        ay1 = jnp.where(sel, jnp.where(valid, by1, zc), ay1)
        ax1 = jnp.where(sel, jnp.where(valid, bx1, zc), ax1)
        ay2 = jnp.where(sel, jnp.where(valid, by2, zc), ay2)
        ax2 = jnp.where(sel, jnp.where(valid, bx2, zc), ax2)
        return ay1, ax1, ay2, ax2

    def group(g, _):
        z = jnp.zeros((B, 128), jnp.float32)
        ay1, ax1, ay2, ax2 = jax.lax.fori_loop(0, 128, step, (z, z, z, z))
        off = g * 128
        oy1_ref[:, pl.ds(off, 128)] = ay1
        ox1_ref[:, pl.ds(off, 128)] = ax1
        oy2_ref[:, pl.ds(off, 128)] = ay2
        ox2_ref[:, pl.ds(off, 128)] = ax2
        return 0

    jax.lax.fori_loop(0, 8, group, 0)


@jax.jit
def kernel(rpn_probs, rpn_bbox, anchors):
    B, N, _ = rpn_probs.shape
    score = rpn_probs[:, :, 1]
    d = [rpn_bbox[:, :, i] for i in range(4)]
    a = [anchors[:, :, i] for i in range(4)]
    f32 = jnp.float32
    outs = pl.pallas_call(
        _nms_kernel,
        out_shape=[jax.ShapeDtypeStruct((B, 1024), f32)] * 4,
        scratch_shapes=[pltpu.VMEM((B, N), f32)] * 6,
    )(score, *d, *a)
    return jnp.stack(outs, axis=-1)[:, :_PROPOSAL_COUNT, :]
